# Initial kernel scaffold; baseline (speedup 1.0000x reference)
import functools

import jax
import jax.numpy as jnp
from jax import lax
from jax.experimental import pallas as pl
from jax.experimental.pallas import tpu as pltpu
from jax.experimental.pallas import tpu_sc as plsc

N = 10000
D = 128
E_EDGES = 320000
EPS = 1e-5
F32 = jnp.float32

NC = 2
NS = 16
NW = NC * NS
CHUNK = 128
CPW = 80
E_PAD = NW * CPW * CHUNK
ROWS_PER_SUB = 632
N_PAD = NS * ROWS_PER_SUB

_PREC = lax.Precision.HIGHEST


def _dot(a, b, dn=None):
  if dn is None:
    return lax.dot(a, b, precision=_PREC, preferred_element_type=F32)
  return lax.dot_general(a, b, dn, precision=_PREC, preferred_element_type=F32)


def _segsum_sc(y, src_c, dst_c, zeros_pad):
  mesh = plsc.VectorSubcoreMesh(core_axis_name="c", subcore_axis_name="s")

  @functools.partial(
      pl.kernel,
      out_type=jax.ShapeDtypeStruct((NC, N_PAD, 32), F32),
      mesh=mesh,
      scratch_types=[
          pltpu.VMEM((CPW, CHUNK), jnp.int32),
          pltpu.VMEM((CPW, CHUNK), jnp.int32),
          pltpu.VMEM((2, CHUNK, 32), F32),
          pltpu.VMEM_SHARED((N_PAD, 32), F32),
          pltpu.SemaphoreType.DMA,
          pltpu.SemaphoreType.DMA,
      ],
  )
  def seg_kernel(y_hbm, src_hbm, dst_hbm, zeros_hbm, out_hbm,
                 src_v, dst_v, rows_v, acc_sh, sem0, sem1):
    c = lax.axis_index("c")
    s = lax.axis_index("s")
    wid = c * NS + s
    pltpu.sync_copy(src_hbm.at[wid], src_v)
    pltpu.sync_copy(dst_hbm.at[wid], dst_v)
    r0 = s * ROWS_PER_SUB
    pltpu.sync_copy(zeros_hbm.at[pl.ds(r0, ROWS_PER_SUB)],
                    acc_sh.at[pl.ds(r0, ROWS_PER_SUB)])
    plsc.subcore_barrier()

    pltpu.async_copy(y_hbm.at[src_v.at[0]], rows_v.at[0], sem0)

    def body(jj, carry):
      j0 = 2 * jj
      pltpu.async_copy(y_hbm.at[src_v.at[j0 + 1]], rows_v.at[1], sem1)
      pltpu.make_async_copy(y_hbm.at[src_v.at[j0]], rows_v.at[0], sem0).wait()
      pltpu.sync_copy(rows_v.at[0], acc_sh.at[dst_v.at[j0]], add=True)

      @pl.when(j0 + 2 < CPW)
      def _():
        pltpu.async_copy(y_hbm.at[src_v.at[j0 + 2]], rows_v.at[0], sem0)

      pltpu.make_async_copy(y_hbm.at[src_v.at[j0 + 1]], rows_v.at[1],
                            sem1).wait()
      pltpu.sync_copy(rows_v.at[1], acc_sh.at[dst_v.at[j0 + 1]], add=True)
      return carry

    lax.fori_loop(0, CPW // 2, body, 0)
    plsc.subcore_barrier()
    pltpu.sync_copy(acc_sh.at[pl.ds(r0, ROWS_PER_SUB)],
                    out_hbm.at[c, pl.ds(r0, ROWS_PER_SUB)])

  return seg_kernel(y, src_c, dst_c, zeros_pad)


def _proj_in(x, wcat):
  def body(x_ref, w_ref, o_ref):
    o_ref[...] = _dot(x_ref[...], w_ref[...])

  return pl.pallas_call(
      body,
      out_shape=jax.ShapeDtypeStruct((N, 64), F32),
  )(x, wcat)


def _gcn_post1(p0, p1, r, brel, g, be, wcat2):
  def body(p0_ref, p1_ref, r_ref, brel_ref, g_ref, be_ref, w_ref, o_ref):
    a = jnp.maximum(p0_ref[...] + p1_ref[...] + brel_ref[...] + r_ref[...],
                    0.0)
    m = jnp.mean(a, axis=0, keepdims=True)
    v = jnp.mean(a * a, axis=0, keepdims=True) - m * m
    h = (a - m) * (g_ref[...] * lax.rsqrt(v + EPS)) + be_ref[...]
    o_ref[...] = _dot(h, w_ref[...])

  return pl.pallas_call(
      body,
      out_shape=jax.ShapeDtypeStruct((N, 64), F32),
  )(p0, p1, r, brel, g, be, wcat2)


def _gcn_post2(p0, p1, r, brel, g, be, w1, b1, gn1, bn1):
  def body(p0_ref, p1_ref, r_ref, brel_ref, g_ref, be_ref, w1_ref, b1_ref,
           gn1_ref, bn1_ref, h_ref, w1f_ref, b1f_ref):
    a = jnp.maximum(p0_ref[...] + p1_ref[...] + brel_ref[...] + r_ref[...],
                    0.0)
    m = jnp.mean(a, axis=0, keepdims=True)
    v = jnp.mean(a * a, axis=0, keepdims=True) - m * m
    h = (a - m) * (g_ref[...] * lax.rsqrt(v + EPS)) + be_ref[...]
    h_ref[...] = h
    mu = jnp.mean(h, axis=0, keepdims=True)
    s_mat = _dot(h, h, (((0,), (0,)), ((), ()))) * (1.0 / N)
    w1 = w1_ref[...]
    muw = _dot(mu, w1)
    sw = _dot(s_mat, w1)
    var = jnp.sum(w1 * sw, axis=0, keepdims=True) - muw * muw
    mean = muw + b1_ref[...]
    alpha = gn1_ref[...] * lax.rsqrt(var + EPS)
    w1f_ref[...] = w1 * alpha
    b1f_ref[...] = (b1_ref[...] - mean) * alpha + bn1_ref[...]

  return pl.pallas_call(
      body,
      out_shape=[
          jax.ShapeDtypeStruct((N, 32), F32),
          jax.ShapeDtypeStruct((32, 2048), F32),
          jax.ShapeDtypeStruct((1, 2048), F32),
      ],
  )(p0, p1, r, brel, g, be, w1, b1, gn1, bn1)


_R1 = 200
_T1 = N // _R1


def _mlp_mid(h2, w1f, b1f, w2, b2):
  def body(h_ref, w1_ref, b1_ref, w2_ref, b2_ref, u2_ref, st_ref, acc):
    i = pl.program_id(0)
    t1 = jnp.maximum(_dot(h_ref[...], w1_ref[...]) + b1_ref[...], 0.0)
    u2 = _dot(t1, w2_ref[...]) + b2_ref[...]
    u2_ref[...] = u2

    @pl.when(i == 0)
    def _():
      acc[...] = jnp.zeros_like(acc)

    acc[0:1, :] += jnp.sum(u2, axis=0, keepdims=True)
    acc[1:2, :] += jnp.sum(u2 * u2, axis=0, keepdims=True)
    st_ref[...] = acc[...]

  return pl.pallas_call(
      body,
      grid=(_T1,),
      in_specs=[
          pl.BlockSpec((_R1, 32), lambda i: (i, 0)),
          pl.BlockSpec((32, 2048), lambda i: (0, 0)),
          pl.BlockSpec((1, 2048), lambda i: (0, 0)),
          pl.BlockSpec((2048, 1024), lambda i: (0, 0)),
          pl.BlockSpec((1, 1024), lambda i: (0, 0)),
      ],
      out_specs=[
          pl.BlockSpec((_R1, 1024), lambda i: (i, 0)),
          pl.BlockSpec((2, 1024), lambda i: (0, 0)),
      ],
      out_shape=[
          jax.ShapeDtypeStruct((N, 1024), F32),
          jax.ShapeDtypeStruct((2, 1024), F32),
      ],
      scratch_shapes=[pltpu.VMEM((2, 1024), F32)],
  )(h2, w1f, b1f, w2, b2)


def _mlp_tail1(u2, st, gn2, bn2, w3, b3):
  def body(u2_ref, st_ref, gn_ref, bn_ref, w3_ref, b3_ref, o_ref):
    m = st_ref[0:1, :] * (1.0 / N)
    v = st_ref[1:2, :] * (1.0 / N) - m * m
    t2 = jnp.maximum((u2_ref[...] - m) * (gn_ref[...] * lax.rsqrt(v + EPS))
                     + bn_ref[...], 0.0)
    o_ref[...] = _dot(t2, w3_ref[...]) + b3_ref[...]

  return pl.pallas_call(
      body,
      grid=(_T1,),
      in_specs=[
          pl.BlockSpec((_R1, 1024), lambda i: (i, 0)),
          pl.BlockSpec((2, 1024), lambda i: (0, 0)),
          pl.BlockSpec((1, 1024), lambda i: (0, 0)),
          pl.BlockSpec((1, 1024), lambda i: (0, 0)),
          pl.BlockSpec((1024, 32), lambda i: (0, 0)),
          pl.BlockSpec((1, 32), lambda i: (0, 0)),
      ],
      out_specs=pl.BlockSpec((_R1, 32), lambda i: (i, 0)),
      out_shape=jax.ShapeDtypeStruct((N, 32), F32),
  )(u2, st, gn2, bn2, w3, b3)


def _mlp_tail2(u3, gn3, bn3, w4, b4):
  def body(u3_ref, gn_ref, bn_ref, w4_ref, b4_ref, o_ref):
    u3 = u3_ref[...]
    m = jnp.mean(u3, axis=0, keepdims=True)
    v = jnp.mean(u3 * u3, axis=0, keepdims=True) - m * m
    t3 = jnp.maximum((u3 - m) * (gn_ref[...] * lax.rsqrt(v + EPS))
                     + bn_ref[...], 0.0)
    o_ref[...] = _dot(t3, w4_ref[...]) + b4_ref[...]

  return pl.pallas_call(
      body,
      out_shape=jax.ShapeDtypeStruct((N, 1), F32),
  )(u3, gn3, bn3, w4, b4)


def kernel(x, edge_index, Wrel1, brel1, Wroot1, Wrel2, brel2, Wroot2,
           g1, be1, g2, be2, W1, b1, gn1, bn1, W2, b2, gn2, bn2,
           W3, b3, gn3, bn3, W4, b4):
  src = edge_index[0]
  dst = edge_index[1]
  npad = E_PAD - E_EDGES
  src_c = jnp.concatenate(
      [src, jnp.zeros((npad,), jnp.int32)]).reshape(NW, CPW, CHUNK)
  dst_c = jnp.concatenate(
      [dst, jnp.full((npad,), N, jnp.int32)]).reshape(NW, CPW, CHUNK)
  zeros_pad = jnp.zeros((N_PAD, 32), F32)

  row = lambda t: t.reshape(1, -1)

  yr = _proj_in(x, jnp.concatenate([Wrel1, Wroot1], axis=1))
  y1 = yr[:, :32]
  r1 = yr[:, 32:]

  seg1 = _segsum_sc(y1, src_c, dst_c, zeros_pad)
  yr2 = _gcn_post1(seg1[0, :N], seg1[1, :N], r1, row(brel1), row(g1),
                   row(be1), jnp.concatenate([Wrel2, Wroot2], axis=1))
  y2 = yr2[:, :32]
  r2 = yr2[:, 32:]

  seg2 = _segsum_sc(y2, src_c, dst_c, zeros_pad)
  h2, w1f, b1f = _gcn_post2(seg2[0, :N], seg2[1, :N], r2, row(brel2),
                            row(g2), row(be2), W1, row(b1), row(gn1),
                            row(bn1))

  u2, st = _mlp_mid(h2, w1f, b1f, W2, row(b2))
  u3 = _mlp_tail1(u2, st, row(gn2), row(bn2), W3, row(b3))
  return _mlp_tail2(u3, row(gn3), row(bn3), W4, row(b4))


# speedup vs baseline: 3.3405x; 3.3405x over previous
import functools

import jax
import jax.numpy as jnp
from jax import lax
from jax.experimental import pallas as pl
from jax.experimental.pallas import tpu as pltpu
from jax.experimental.pallas import tpu_sc as plsc

N = 10000
D = 128
E_EDGES = 320000
EPS = 1e-5
F32 = jnp.float32

NC = 2
NS = 16
NW = NC * NS
CHUNK = 128
CPW = 80
E_PAD = NW * CPW * CHUNK
ROWS_PER_SUB = 632
N_PAD = NS * ROWS_PER_SUB

_DEF = lax.Precision.DEFAULT


def _dot(a, b):
  return lax.dot(a, b, precision=_DEF, preferred_element_type=F32)


def _segsum_sc(y, src_c, dst_c, zeros_pad, feat):
  mesh = plsc.VectorSubcoreMesh(core_axis_name="c", subcore_axis_name="s")

  @functools.partial(
      pl.kernel,
      out_type=jax.ShapeDtypeStruct((NC, N_PAD, feat), F32),
      mesh=mesh,
      scratch_types=[
          pltpu.VMEM((CPW, CHUNK), jnp.int32),
          pltpu.VMEM((CPW, CHUNK), jnp.int32),
          pltpu.VMEM((2, CHUNK, feat), F32),
          pltpu.VMEM_SHARED((N_PAD, feat), F32),
          pltpu.SemaphoreType.DMA,
          pltpu.SemaphoreType.DMA,
      ],
      compiler_params=pltpu.CompilerParams(use_tc_tiling_on_sc=False),
  )
  def seg_kernel(y_hbm, src_hbm, dst_hbm, zeros_hbm, out_hbm,
                 src_v, dst_v, rows_v, acc_sh, sem0, sem1):
    c = lax.axis_index("c")
    s = lax.axis_index("s")
    wid = c * NS + s
    pltpu.sync_copy(src_hbm.at[wid], src_v)
    pltpu.sync_copy(dst_hbm.at[wid], dst_v)
    r0 = s * ROWS_PER_SUB
    pltpu.sync_copy(zeros_hbm.at[pl.ds(r0, ROWS_PER_SUB)],
                    acc_sh.at[pl.ds(r0, ROWS_PER_SUB)])
    plsc.subcore_barrier()

    pltpu.async_copy(y_hbm.at[src_v.at[0]], rows_v.at[0], sem0)

    def body(jj, carry):
      j0 = 2 * jj
      pltpu.async_copy(y_hbm.at[src_v.at[j0 + 1]], rows_v.at[1], sem1)
      pltpu.make_async_copy(y_hbm.at[src_v.at[j0]], rows_v.at[0], sem0).wait()
      pltpu.sync_copy(rows_v.at[0], acc_sh.at[dst_v.at[j0]], add=True)

      @pl.when(j0 + 2 < CPW)
      def _():
        pltpu.async_copy(y_hbm.at[src_v.at[j0 + 2]], rows_v.at[0], sem0)

      pltpu.make_async_copy(y_hbm.at[src_v.at[j0 + 1]], rows_v.at[1],
                            sem1).wait()
      pltpu.sync_copy(rows_v.at[1], acc_sh.at[dst_v.at[j0 + 1]], add=True)
      return carry

    lax.fori_loop(0, CPW // 2, body, 0)
    plsc.subcore_barrier()
    pltpu.sync_copy(acc_sh.at[pl.ds(r0, ROWS_PER_SUB)],
                    out_hbm.at[c, pl.ds(r0, ROWS_PER_SUB)])

  return seg_kernel(y, src_c, dst_c, zeros_pad)


def _gcn_layer1(a0, a1, b0, b1, h, wrel_lo, wrel_hi, brel, wroot, g, be):
  def body(a0_ref, a1_ref, b0_ref, b1_ref, h_ref, wlo_ref, whi_ref,
           brel_ref, wr_ref, g_ref, be_ref, o_ref):
    agg_lo = a0_ref[...] + a1_ref[...]
    agg_hi = b0_ref[...] + b1_ref[...]
    p = (_dot(agg_lo, wlo_ref[...]) + _dot(agg_hi, whi_ref[...])
         + brel_ref[...] + _dot(h_ref[...], wr_ref[...]))
    a = jnp.maximum(p, 0.0)
    m = jnp.mean(a, axis=0, keepdims=True)
    v = jnp.mean(a * a, axis=0, keepdims=True) - m * m
    o_ref[...] = (a - m) * (g_ref[...] * lax.rsqrt(v + EPS)) + be_ref[...]

  return pl.pallas_call(
      body,
      out_shape=jax.ShapeDtypeStruct((N, 32), F32),
  )(a0, a1, b0, b1, h, wrel_lo, wrel_hi, brel, wroot, g, be)


def _gcn_layer2(a0, a1, h, wrel, brel, wroot, g, be):
  def body(a0_ref, a1_ref, h_ref, wl_ref, brel_ref, wr_ref, g_ref, be_ref,
           o_ref):
    agg = a0_ref[...] + a1_ref[...]
    p = _dot(agg, wl_ref[...]) + brel_ref[...] + _dot(h_ref[...], wr_ref[...])
    a = jnp.maximum(p, 0.0)
    m = jnp.mean(a, axis=0, keepdims=True)
    v = jnp.mean(a * a, axis=0, keepdims=True) - m * m
    o_ref[...] = (a - m) * (g_ref[...] * lax.rsqrt(v + EPS)) + be_ref[...]

  return pl.pallas_call(
      body,
      out_shape=jax.ShapeDtypeStruct((N, 32), F32),
  )(a0, a1, h, wrel, brel, wroot, g, be)


_R1 = 200
_T1 = N // _R1


def _matmul_stats(a, w, b, k_in, k_out):
  def body(a_ref, w_ref, b_ref, u_ref, st_ref, acc):
    i = pl.program_id(0)
    u = _dot(a_ref[...], w_ref[...]) + b_ref[...]
    u_ref[...] = u

    @pl.when(i == 0)
    def _():
      acc[...] = jnp.zeros_like(acc)

    acc[0:1, :] += jnp.sum(u, axis=0, keepdims=True)
    acc[1:2, :] += jnp.sum(u * u, axis=0, keepdims=True)
    st_ref[...] = acc[...]

  return pl.pallas_call(
      body,
      grid=(_T1,),
      in_specs=[
          pl.BlockSpec((_R1, k_in), lambda i: (i, 0)),
          pl.BlockSpec((k_in, k_out), lambda i: (0, 0)),
          pl.BlockSpec((1, k_out), lambda i: (0, 0)),
      ],
      out_specs=[
          pl.BlockSpec((_R1, k_out), lambda i: (i, 0)),
          pl.BlockSpec((2, k_out), lambda i: (0, 0)),
      ],
      out_shape=[
          jax.ShapeDtypeStruct((N, k_out), F32),
          jax.ShapeDtypeStruct((2, k_out), F32),
      ],
      scratch_shapes=[pltpu.VMEM((2, k_out), F32)],
  )(a, w, b)


def _bn_relu_matmul_stats(u, st, g, be, w, b, k_in, k_out):
  def body(u_ref, st_ref, g_ref, be_ref, w_ref, b_ref, o_ref, so_ref, acc):
    i = pl.program_id(0)
    m = st_ref[0:1, :] * (1.0 / N)
    var = st_ref[1:2, :] * (1.0 / N) - m * m
    t = jnp.maximum((u_ref[...] - m) * (g_ref[...] * lax.rsqrt(var + EPS))
                    + be_ref[...], 0.0)
    o = _dot(t, w_ref[...]) + b_ref[...]
    o_ref[...] = o

    @pl.when(i == 0)
    def _():
      acc[...] = jnp.zeros_like(acc)

    acc[0:1, :] += jnp.sum(o, axis=0, keepdims=True)
    acc[1:2, :] += jnp.sum(o * o, axis=0, keepdims=True)
    so_ref[...] = acc[...]

  return pl.pallas_call(
      body,
      grid=(_T1,),
      in_specs=[
          pl.BlockSpec((_R1, k_in), lambda i: (i, 0)),
          pl.BlockSpec((2, k_in), lambda i: (0, 0)),
          pl.BlockSpec((1, k_in), lambda i: (0, 0)),
          pl.BlockSpec((1, k_in), lambda i: (0, 0)),
          pl.BlockSpec((k_in, k_out), lambda i: (0, 0)),
          pl.BlockSpec((1, k_out), lambda i: (0, 0)),
      ],
      out_specs=[
          pl.BlockSpec((_R1, k_out), lambda i: (i, 0)),
          pl.BlockSpec((2, k_out), lambda i: (0, 0)),
      ],
      out_shape=[
          jax.ShapeDtypeStruct((N, k_out), F32),
          jax.ShapeDtypeStruct((2, k_out), F32),
      ],
      scratch_shapes=[pltpu.VMEM((2, k_out), F32)],
  )(u, st, g, be, w, b)


def _mlp_tail(u3, st3, gn3, bn3, w4, b4):
  def body(u3_ref, st_ref, gn_ref, bn_ref, w4_ref, b4_ref, o_ref):
    m = st_ref[0:1, :] * (1.0 / N)
    v = st_ref[1:2, :] * (1.0 / N) - m * m
    t3 = jnp.maximum((u3_ref[...] - m) * (gn_ref[...] * lax.rsqrt(v + EPS))
                     + bn_ref[...], 0.0)
    o_ref[...] = _dot(t3, w4_ref[...]) + b4_ref[...]

  return pl.pallas_call(
      body,
      out_shape=jax.ShapeDtypeStruct((N, 1), F32),
  )(u3, st3, gn3, bn3, w4, b4)


def kernel(x, edge_index, Wrel1, brel1, Wroot1, Wrel2, brel2, Wroot2,
           g1, be1, g2, be2, W1, b1, gn1, bn1, W2, b2, gn2, bn2,
           W3, b3, gn3, bn3, W4, b4):
  src = edge_index[0]
  dst = edge_index[1]
  npad = E_PAD - E_EDGES
  src_c = jnp.concatenate(
      [src, jnp.zeros((npad,), jnp.int32)]).reshape(NW, CPW, CHUNK)
  dst_c = jnp.concatenate(
      [dst, jnp.full((npad,), N, jnp.int32)]).reshape(NW, CPW, CHUNK)
  zeros64 = jnp.zeros((N_PAD, 64), F32)
  zeros32 = jnp.zeros((N_PAD, 32), F32)

  row = lambda t: t.reshape(1, -1)

  seg1a = _segsum_sc(x[:, :64], src_c, dst_c, zeros64, 64)
  seg1b = _segsum_sc(x[:, 64:], src_c, dst_c, zeros64, 64)
  h1 = _gcn_layer1(seg1a[0, :N], seg1a[1, :N], seg1b[0, :N], seg1b[1, :N],
                   x, Wrel1[:64], Wrel1[64:], row(brel1), Wroot1,
                   row(g1), row(be1))

  seg2 = _segsum_sc(h1, src_c, dst_c, zeros32, 32)
  h2 = _gcn_layer2(seg2[0, :N], seg2[1, :N], h1, Wrel2, row(brel2), Wroot2,
                   row(g2), row(be2))

  u1, st1 = _matmul_stats(h2, W1, row(b1), 32, 2048)
  u2, st2 = _bn_relu_matmul_stats(u1, st1, row(gn1), row(bn1), W2, row(b2),
                                  2048, 1024)
  u3, st3 = _bn_relu_matmul_stats(u2, st2, row(gn2), row(bn2), W3, row(b3),
                                  1024, 32)
  return _mlp_tail(u3, st3, row(gn3), row(bn3), W4, row(b4))


# speedup vs baseline: 6.1245x; 1.8334x over previous
import functools

import jax
import jax.numpy as jnp
from jax import lax
from jax.experimental import pallas as pl
from jax.experimental.pallas import tpu as pltpu
from jax.experimental.pallas import tpu_sc as plsc

N = 10000
D = 128
E_EDGES = 320000
EPS = 1e-5
F32 = jnp.float32

NC = 2
NS = 16
NW = NC * NS
CHUNK = 128
CPW = 80
E_PAD = NW * CPW * CHUNK
ROWS_PER_SUB = 632
N_PAD = NS * ROWS_PER_SUB

_DEF = lax.Precision.DEFAULT


def _dot(a, b):
  return lax.dot(a, b, precision=_DEF, preferred_element_type=F32)


TBL_PER_SUB = N // NS


def _segsum_sc(y, src_c, dst_c, zeros_pad, feat):
  mesh = plsc.VectorSubcoreMesh(core_axis_name="c", subcore_axis_name="s")

  @functools.partial(
      pl.kernel,
      out_type=jax.ShapeDtypeStruct((NC, N_PAD, feat), F32),
      mesh=mesh,
      scratch_types=[
          pltpu.VMEM((CPW, CHUNK), jnp.int32),
          pltpu.VMEM((CPW, CHUNK), jnp.int32),
          pltpu.VMEM((2, CHUNK, feat), F32),
          pltpu.VMEM_SHARED((N_PAD, feat), F32),
          pltpu.VMEM_SHARED((N, feat), F32),
          pltpu.SemaphoreType.DMA,
          pltpu.SemaphoreType.DMA,
      ],
      compiler_params=pltpu.CompilerParams(use_tc_tiling_on_sc=False),
  )
  def seg_kernel(y_hbm, src_hbm, dst_hbm, zeros_hbm, out_hbm,
                 src_v, dst_v, rows_v, acc_sh, tbl_sh, sem0, sem1):
    c = lax.axis_index("c")
    s = lax.axis_index("s")
    wid = c * NS + s
    pltpu.sync_copy(src_hbm.at[wid], src_v)
    pltpu.sync_copy(dst_hbm.at[wid], dst_v)
    t0 = s * TBL_PER_SUB
    pltpu.sync_copy(y_hbm.at[pl.ds(t0, TBL_PER_SUB)],
                    tbl_sh.at[pl.ds(t0, TBL_PER_SUB)])
    r0 = s * ROWS_PER_SUB
    pltpu.sync_copy(zeros_hbm.at[pl.ds(r0, ROWS_PER_SUB)],
                    acc_sh.at[pl.ds(r0, ROWS_PER_SUB)])
    plsc.subcore_barrier()

    pltpu.async_copy(tbl_sh.at[src_v.at[0]], rows_v.at[0], sem0)

    def body(jj, carry):
      j0 = 2 * jj
      pltpu.async_copy(tbl_sh.at[src_v.at[j0 + 1]], rows_v.at[1], sem1)
      pltpu.make_async_copy(tbl_sh.at[src_v.at[j0]], rows_v.at[0], sem0).wait()
      pltpu.sync_copy(rows_v.at[0], acc_sh.at[dst_v.at[j0]], add=True)

      @pl.when(j0 + 2 < CPW)
      def _():
        pltpu.async_copy(tbl_sh.at[src_v.at[j0 + 2]], rows_v.at[0], sem0)

      pltpu.make_async_copy(tbl_sh.at[src_v.at[j0 + 1]], rows_v.at[1],
                            sem1).wait()
      pltpu.sync_copy(rows_v.at[1], acc_sh.at[dst_v.at[j0 + 1]], add=True)
      return carry

    lax.fori_loop(0, CPW // 2, body, 0)
    plsc.subcore_barrier()
    pltpu.sync_copy(acc_sh.at[pl.ds(r0, ROWS_PER_SUB)],
                    out_hbm.at[c, pl.ds(r0, ROWS_PER_SUB)])

  return seg_kernel(y, src_c, dst_c, zeros_pad)


def _gcn_layer1(a0, a1, b0, b1, c0, c1, h, wrel_a, wrel_b, wrel_c,
                brel, wroot, g, be):
  def body(a0_ref, a1_ref, b0_ref, b1_ref, c0_ref, c1_ref, h_ref,
           wa_ref, wb_ref, wc_ref, brel_ref, wr_ref, g_ref, be_ref, o_ref):
    p = (_dot(a0_ref[...] + a1_ref[...], wa_ref[...])
         + _dot(b0_ref[...] + b1_ref[...], wb_ref[...])
         + _dot(c0_ref[...] + c1_ref[...], wc_ref[...])
         + brel_ref[...] + _dot(h_ref[...], wr_ref[...]))
    a = jnp.maximum(p, 0.0)
    m = jnp.mean(a, axis=0, keepdims=True)
    v = jnp.mean(a * a, axis=0, keepdims=True) - m * m
    o_ref[...] = (a - m) * (g_ref[...] * lax.rsqrt(v + EPS)) + be_ref[...]

  return pl.pallas_call(
      body,
      out_shape=jax.ShapeDtypeStruct((N, 32), F32),
  )(a0, a1, b0, b1, c0, c1, h, wrel_a, wrel_b, wrel_c, brel, wroot, g, be)


def _gcn_layer2(a0, a1, h, wrel, brel, wroot, g, be):
  def body(a0_ref, a1_ref, h_ref, wl_ref, brel_ref, wr_ref, g_ref, be_ref,
           o_ref):
    agg = a0_ref[...] + a1_ref[...]
    p = _dot(agg, wl_ref[...]) + brel_ref[...] + _dot(h_ref[...], wr_ref[...])
    a = jnp.maximum(p, 0.0)
    m = jnp.mean(a, axis=0, keepdims=True)
    v = jnp.mean(a * a, axis=0, keepdims=True) - m * m
    o_ref[...] = (a - m) * (g_ref[...] * lax.rsqrt(v + EPS)) + be_ref[...]

  return pl.pallas_call(
      body,
      out_shape=jax.ShapeDtypeStruct((N, 32), F32),
  )(a0, a1, h, wrel, brel, wroot, g, be)


_R1 = 200
_T1 = N // _R1


def _matmul_stats(a, w, b, k_in, k_out):
  def body(a_ref, w_ref, b_ref, u_ref, st_ref, acc):
    i = pl.program_id(0)
    u = _dot(a_ref[...], w_ref[...]) + b_ref[...]
    u_ref[...] = u

    @pl.when(i == 0)
    def _():
      acc[...] = jnp.zeros_like(acc)

    acc[0:1, :] += jnp.sum(u, axis=0, keepdims=True)
    acc[1:2, :] += jnp.sum(u * u, axis=0, keepdims=True)
    st_ref[...] = acc[...]

  return pl.pallas_call(
      body,
      grid=(_T1,),
      in_specs=[
          pl.BlockSpec((_R1, k_in), lambda i: (i, 0)),
          pl.BlockSpec((k_in, k_out), lambda i: (0, 0)),
          pl.BlockSpec((1, k_out), lambda i: (0, 0)),
      ],
      out_specs=[
          pl.BlockSpec((_R1, k_out), lambda i: (i, 0)),
          pl.BlockSpec((2, k_out), lambda i: (0, 0)),
      ],
      out_shape=[
          jax.ShapeDtypeStruct((N, k_out), F32),
          jax.ShapeDtypeStruct((2, k_out), F32),
      ],
      scratch_shapes=[pltpu.VMEM((2, k_out), F32)],
  )(a, w, b)


def _bn_relu_matmul_stats(u, st, g, be, w, b, k_in, k_out):
  def body(u_ref, st_ref, g_ref, be_ref, w_ref, b_ref, o_ref, so_ref, acc):
    i = pl.program_id(0)
    m = st_ref[0:1, :] * (1.0 / N)
    var = st_ref[1:2, :] * (1.0 / N) - m * m
    t = jnp.maximum((u_ref[...] - m) * (g_ref[...] * lax.rsqrt(var + EPS))
                    + be_ref[...], 0.0)
    o = _dot(t, w_ref[...]) + b_ref[...]
    o_ref[...] = o

    @pl.when(i == 0)
    def _():
      acc[...] = jnp.zeros_like(acc)

    acc[0:1, :] += jnp.sum(o, axis=0, keepdims=True)
    acc[1:2, :] += jnp.sum(o * o, axis=0, keepdims=True)
    so_ref[...] = acc[...]

  return pl.pallas_call(
      body,
      grid=(_T1,),
      in_specs=[
          pl.BlockSpec((_R1, k_in), lambda i: (i, 0)),
          pl.BlockSpec((2, k_in), lambda i: (0, 0)),
          pl.BlockSpec((1, k_in), lambda i: (0, 0)),
          pl.BlockSpec((1, k_in), lambda i: (0, 0)),
          pl.BlockSpec((k_in, k_out), lambda i: (0, 0)),
          pl.BlockSpec((1, k_out), lambda i: (0, 0)),
      ],
      out_specs=[
          pl.BlockSpec((_R1, k_out), lambda i: (i, 0)),
          pl.BlockSpec((2, k_out), lambda i: (0, 0)),
      ],
      out_shape=[
          jax.ShapeDtypeStruct((N, k_out), F32),
          jax.ShapeDtypeStruct((2, k_out), F32),
      ],
      scratch_shapes=[pltpu.VMEM((2, k_out), F32)],
  )(u, st, g, be, w, b)


def _mlp_tail(u3, st3, gn3, bn3, w4, b4):
  def body(u3_ref, st_ref, gn_ref, bn_ref, w4_ref, b4_ref, o_ref):
    m = st_ref[0:1, :] * (1.0 / N)
    v = st_ref[1:2, :] * (1.0 / N) - m * m
    t3 = jnp.maximum((u3_ref[...] - m) * (gn_ref[...] * lax.rsqrt(v + EPS))
                     + bn_ref[...], 0.0)
    o_ref[...] = _dot(t3, w4_ref[...]) + b4_ref[...]

  return pl.pallas_call(
      body,
      out_shape=jax.ShapeDtypeStruct((N, 1), F32),
  )(u3, st3, gn3, bn3, w4, b4)


def kernel(x, edge_index, Wrel1, brel1, Wroot1, Wrel2, brel2, Wroot2,
           g1, be1, g2, be2, W1, b1, gn1, bn1, W2, b2, gn2, bn2,
           W3, b3, gn3, bn3, W4, b4):
  src = edge_index[0]
  dst = edge_index[1]
  npad = E_PAD - E_EDGES
  src_c = jnp.concatenate(
      [src, jnp.zeros((npad,), jnp.int32)]).reshape(NW, CPW, CHUNK)
  dst_c = jnp.concatenate(
      [dst, jnp.full((npad,), N, jnp.int32)]).reshape(NW, CPW, CHUNK)
  zeros48 = jnp.zeros((N_PAD, 48), F32)
  zeros32 = jnp.zeros((N_PAD, 32), F32)

  row = lambda t: t.reshape(1, -1)

  seg1a = _segsum_sc(x[:, :48], src_c, dst_c, zeros48, 48)
  seg1b = _segsum_sc(x[:, 48:96], src_c, dst_c, zeros48, 48)
  seg1c = _segsum_sc(x[:, 96:], src_c, dst_c, zeros32, 32)
  h1 = _gcn_layer1(seg1a[0, :N], seg1a[1, :N], seg1b[0, :N], seg1b[1, :N],
                   seg1c[0, :N], seg1c[1, :N],
                   x, Wrel1[:48], Wrel1[48:96], Wrel1[96:], row(brel1),
                   Wroot1, row(g1), row(be1))

  seg2 = _segsum_sc(h1, src_c, dst_c, zeros32, 32)
  h2 = _gcn_layer2(seg2[0, :N], seg2[1, :N], h1, Wrel2, row(brel2), Wroot2,
                   row(g2), row(be2))

  u1, st1 = _matmul_stats(h2, W1, row(b1), 32, 2048)
  u2, st2 = _bn_relu_matmul_stats(u1, st1, row(gn1), row(bn1), W2, row(b2),
                                  2048, 1024)
  u3, st3 = _bn_relu_matmul_stats(u2, st2, row(gn2), row(bn2), W3, row(b3),
                                  1024, 32)
  return _mlp_tail(u3, st3, row(gn3), row(bn3), W4, row(b4))


# speedup vs baseline: 6.4705x; 1.0565x over previous
import functools

import jax
import jax.numpy as jnp
from jax import lax
from jax.experimental import pallas as pl
from jax.experimental.pallas import tpu as pltpu
from jax.experimental.pallas import tpu_sc as plsc

N = 10000
D = 128
E_EDGES = 320000
EPS = 1e-5
F32 = jnp.float32

NC = 2
NS = 16
NW = NC * NS
CHUNK = 128
CPW = 80
E_PAD = NW * CPW * CHUNK
ROWS_PER_SUB = 632
N_PAD = NS * ROWS_PER_SUB

_DEF = lax.Precision.DEFAULT


def _dot(a, b):
  return lax.dot(a, b, precision=_DEF, preferred_element_type=F32)


TBL_PER_SUB = N // NS


def _segsum_sc(y, src_c, dst_c, zeros_pad, feat):
  mesh = plsc.VectorSubcoreMesh(core_axis_name="c", subcore_axis_name="s")

  @functools.partial(
      pl.kernel,
      out_type=jax.ShapeDtypeStruct((NC, N_PAD, feat), F32),
      mesh=mesh,
      scratch_types=[
          pltpu.VMEM((CPW, CHUNK), jnp.int32),
          pltpu.VMEM((CPW, CHUNK), jnp.int32),
          pltpu.VMEM((2, CHUNK, feat), F32),
          pltpu.VMEM_SHARED((N_PAD, feat), F32),
          pltpu.VMEM_SHARED((N, feat), F32),
          pltpu.SemaphoreType.DMA,
          pltpu.SemaphoreType.DMA,
      ],
      compiler_params=pltpu.CompilerParams(use_tc_tiling_on_sc=False),
  )
  def seg_kernel(y_hbm, src_hbm, dst_hbm, zeros_hbm, out_hbm,
                 src_v, dst_v, rows_v, acc_sh, tbl_sh, sem0, sem1):
    c = lax.axis_index("c")
    s = lax.axis_index("s")
    wid = c * NS + s
    pltpu.sync_copy(src_hbm.at[wid], src_v)
    pltpu.sync_copy(dst_hbm.at[wid], dst_v)
    t0 = s * TBL_PER_SUB
    pltpu.sync_copy(y_hbm.at[pl.ds(t0, TBL_PER_SUB)],
                    tbl_sh.at[pl.ds(t0, TBL_PER_SUB)])
    r0 = s * ROWS_PER_SUB
    pltpu.sync_copy(zeros_hbm.at[pl.ds(r0, ROWS_PER_SUB)],
                    acc_sh.at[pl.ds(r0, ROWS_PER_SUB)])
    plsc.subcore_barrier()

    pltpu.async_copy(tbl_sh.at[src_v.at[0]], rows_v.at[0], sem0)

    def body(jj, carry):
      j0 = 2 * jj
      pltpu.async_copy(tbl_sh.at[src_v.at[j0 + 1]], rows_v.at[1], sem1)
      pltpu.make_async_copy(tbl_sh.at[src_v.at[j0]], rows_v.at[0], sem0).wait()
      pltpu.sync_copy(rows_v.at[0], acc_sh.at[dst_v.at[j0]], add=True)

      @pl.when(j0 + 2 < CPW)
      def _():
        pltpu.async_copy(tbl_sh.at[src_v.at[j0 + 2]], rows_v.at[0], sem0)

      pltpu.make_async_copy(tbl_sh.at[src_v.at[j0 + 1]], rows_v.at[1],
                            sem1).wait()
      pltpu.sync_copy(rows_v.at[1], acc_sh.at[dst_v.at[j0 + 1]], add=True)
      return carry

    lax.fori_loop(0, CPW // 2, body, 0)
    plsc.subcore_barrier()
    pltpu.sync_copy(acc_sh.at[pl.ds(r0, ROWS_PER_SUB)],
                    out_hbm.at[c, pl.ds(r0, ROWS_PER_SUB)])

  return seg_kernel(y, src_c, dst_c, zeros_pad)


def _gcn_layer1(a0, a1, b0, b1, c0, c1, h, wrel_a, wrel_b, wrel_c,
                brel, wroot, g, be):
  def body(a0_ref, a1_ref, b0_ref, b1_ref, c0_ref, c1_ref, h_ref,
           wa_ref, wb_ref, wc_ref, brel_ref, wr_ref, g_ref, be_ref, o_ref):
    p = (_dot(a0_ref[...] + a1_ref[...], wa_ref[...])
         + _dot(b0_ref[...] + b1_ref[...], wb_ref[...])
         + _dot(c0_ref[...] + c1_ref[...], wc_ref[...])
         + brel_ref[...] + _dot(h_ref[...], wr_ref[...]))
    a = jnp.maximum(p, 0.0)
    m = jnp.mean(a, axis=0, keepdims=True)
    v = jnp.mean(a * a, axis=0, keepdims=True) - m * m
    o_ref[...] = (a - m) * (g_ref[...] * lax.rsqrt(v + EPS)) + be_ref[...]

  return pl.pallas_call(
      body,
      out_shape=jax.ShapeDtypeStruct((N, 32), F32),
  )(a0, a1, b0, b1, c0, c1, h, wrel_a, wrel_b, wrel_c, brel, wroot, g, be)


def _gcn_layer2_stats(a0, a1, h, wrel, brel, wroot, g, be, w1, b1):
  def body(a0_ref, a1_ref, h_ref, wl_ref, brel_ref, wr_ref, g_ref, be_ref,
           w1_ref, b1_ref, h2_ref, st_ref):
    agg = a0_ref[...] + a1_ref[...]
    p = _dot(agg, wl_ref[...]) + brel_ref[...] + _dot(h_ref[...], wr_ref[...])
    a = jnp.maximum(p, 0.0)
    m = jnp.mean(a, axis=0, keepdims=True)
    v = jnp.mean(a * a, axis=0, keepdims=True) - m * m
    h2 = (a - m) * (g_ref[...] * lax.rsqrt(v + EPS)) + be_ref[...]
    h2_ref[...] = h2
    blocks = []
    for k in range(8):
      sl = slice(256 * k, 256 * (k + 1))
      u = _dot(h2, w1_ref[:, sl]) + b1_ref[:, sl]
      blocks.append(jnp.concatenate(
          [jnp.sum(u, axis=0, keepdims=True),
           jnp.sum(u * u, axis=0, keepdims=True)], axis=0))
    st_ref[...] = jnp.concatenate(blocks, axis=1)

  return pl.pallas_call(
      body,
      out_shape=[
          jax.ShapeDtypeStruct((N, 32), F32),
          jax.ShapeDtypeStruct((2, 2048), F32),
      ],
  )(a0, a1, h, wrel, brel, wroot, g, be, w1, b1)


_R1 = 200
_T1 = N // _R1


def _recompute_bn_relu_matmul_stats(h2, w1, b1, st1, gn1, bn1, w2, b2):
  def body(h2_ref, w1_ref, b1_ref, st_ref, g_ref, be_ref, w2_ref, b2_ref,
           o_ref, so_ref, acc):
    i = pl.program_id(0)
    u1 = _dot(h2_ref[...], w1_ref[...]) + b1_ref[...]
    m = st_ref[0:1, :] * (1.0 / N)
    var = st_ref[1:2, :] * (1.0 / N) - m * m
    t = jnp.maximum((u1 - m) * (g_ref[...] * lax.rsqrt(var + EPS))
                    + be_ref[...], 0.0)
    o = _dot(t, w2_ref[...]) + b2_ref[...]
    o_ref[...] = o

    @pl.when(i == 0)
    def _():
      acc[...] = jnp.zeros_like(acc)

    acc[0:1, :] += jnp.sum(o, axis=0, keepdims=True)
    acc[1:2, :] += jnp.sum(o * o, axis=0, keepdims=True)
    so_ref[...] = acc[...]

  return pl.pallas_call(
      body,
      grid=(_T1,),
      in_specs=[
          pl.BlockSpec((_R1, 32), lambda i: (i, 0)),
          pl.BlockSpec((32, 2048), lambda i: (0, 0)),
          pl.BlockSpec((1, 2048), lambda i: (0, 0)),
          pl.BlockSpec((2, 2048), lambda i: (0, 0)),
          pl.BlockSpec((1, 2048), lambda i: (0, 0)),
          pl.BlockSpec((1, 2048), lambda i: (0, 0)),
          pl.BlockSpec((2048, 1024), lambda i: (0, 0)),
          pl.BlockSpec((1, 1024), lambda i: (0, 0)),
      ],
      out_specs=[
          pl.BlockSpec((_R1, 1024), lambda i: (i, 0)),
          pl.BlockSpec((2, 1024), lambda i: (0, 0)),
      ],
      out_shape=[
          jax.ShapeDtypeStruct((N, 1024), F32),
          jax.ShapeDtypeStruct((2, 1024), F32),
      ],
      scratch_shapes=[pltpu.VMEM((2, 1024), F32)],
  )(h2, w1, b1, st1, gn1, bn1, w2, b2)


def _stage3_tail(u2, st2, gn2, bn2, w3, b3, gn3, bn3, w4, b4):
  def body(u2_ref, st2_ref, g2_ref, be2_ref, w3_ref, b3_ref, g3_ref, be3_ref,
           w4_ref, b4_ref, o_ref, u3_scr, acc):
    p = pl.program_id(0)
    i = pl.program_id(1)

    @pl.when(jnp.logical_and(p == 0, i == 0))
    def _():
      acc[...] = jnp.zeros_like(acc)

    @pl.when(p == 0)
    def _():
      m = st2_ref[0:1, :] * (1.0 / N)
      var = st2_ref[1:2, :] * (1.0 / N) - m * m
      t = jnp.maximum((u2_ref[...] - m) * (g2_ref[...] * lax.rsqrt(var + EPS))
                      + be2_ref[...], 0.0)
      u3 = _dot(t, w3_ref[...]) + b3_ref[...]
      u3_scr[pl.ds(i * _R1, _R1), :] = u3
      acc[0:1, :] += jnp.sum(u3, axis=0, keepdims=True)
      acc[1:2, :] += jnp.sum(u3 * u3, axis=0, keepdims=True)

    @pl.when(p == 1)
    def _():
      m3 = acc[0:1, :] * (1.0 / N)
      v3 = acc[1:2, :] * (1.0 / N) - m3 * m3
      t3 = jnp.maximum(
          (u3_scr[pl.ds(i * _R1, _R1), :] - m3)
          * (g3_ref[...] * lax.rsqrt(v3 + EPS)) + be3_ref[...], 0.0)
      o_ref[...] = _dot(t3, w4_ref[...]) + b4_ref[...]

  return pl.pallas_call(
      body,
      grid=(2, _T1),
      in_specs=[
          pl.BlockSpec((_R1, 1024), lambda p, i: (jnp.where(p == 0, i, 0), 0)),
          pl.BlockSpec((2, 1024), lambda p, i: (0, 0)),
          pl.BlockSpec((1, 1024), lambda p, i: (0, 0)),
          pl.BlockSpec((1, 1024), lambda p, i: (0, 0)),
          pl.BlockSpec((1024, 32), lambda p, i: (0, 0)),
          pl.BlockSpec((1, 32), lambda p, i: (0, 0)),
          pl.BlockSpec((1, 32), lambda p, i: (0, 0)),
          pl.BlockSpec((1, 32), lambda p, i: (0, 0)),
          pl.BlockSpec((32, 1), lambda p, i: (0, 0)),
          pl.BlockSpec((1, 1), lambda p, i: (0, 0)),
      ],
      out_specs=pl.BlockSpec((_R1, 1), lambda p, i: (i, 0)),
      out_shape=jax.ShapeDtypeStruct((N, 1), F32),
      scratch_shapes=[
          pltpu.VMEM((N, 32), F32),
          pltpu.VMEM((2, 32), F32),
      ],
  )(u2, st2, gn2, bn2, w3, b3, gn3, bn3, w4, b4)


def kernel(x, edge_index, Wrel1, brel1, Wroot1, Wrel2, brel2, Wroot2,
           g1, be1, g2, be2, W1, b1, gn1, bn1, W2, b2, gn2, bn2,
           W3, b3, gn3, bn3, W4, b4):
  src = edge_index[0]
  dst = edge_index[1]
  npad = E_PAD - E_EDGES
  src_c = jnp.concatenate(
      [src, jnp.zeros((npad,), jnp.int32)]).reshape(NW, CPW, CHUNK)
  dst_c = jnp.concatenate(
      [dst, jnp.full((npad,), N, jnp.int32)]).reshape(NW, CPW, CHUNK)
  zeros48 = jnp.zeros((N_PAD, 48), F32)
  zeros32 = jnp.zeros((N_PAD, 32), F32)

  row = lambda t: t.reshape(1, -1)

  seg1a = _segsum_sc(x[:, :48], src_c, dst_c, zeros48, 48)
  seg1b = _segsum_sc(x[:, 48:96], src_c, dst_c, zeros48, 48)
  seg1c = _segsum_sc(x[:, 96:], src_c, dst_c, zeros32, 32)
  h1 = _gcn_layer1(seg1a[0, :N], seg1a[1, :N], seg1b[0, :N], seg1b[1, :N],
                   seg1c[0, :N], seg1c[1, :N],
                   x, Wrel1[:48], Wrel1[48:96], Wrel1[96:], row(brel1),
                   Wroot1, row(g1), row(be1))

  seg2 = _segsum_sc(h1, src_c, dst_c, zeros32, 32)
  h2, st1 = _gcn_layer2_stats(seg2[0, :N], seg2[1, :N], h1, Wrel2,
                              row(brel2), Wroot2, row(g2), row(be2),
                              W1, row(b1))

  u2, st2 = _recompute_bn_relu_matmul_stats(h2, W1, row(b1), st1,
                                            row(gn1), row(bn1), W2, row(b2))
  return _stage3_tail(u2, st2, row(gn2), row(bn2), W3, row(b3),
                      row(gn3), row(bn3), W4, row(b4))


# speedup vs baseline: 6.6740x; 1.0315x over previous
import functools

import jax
import jax.numpy as jnp
from jax import lax
from jax.experimental import pallas as pl
from jax.experimental.pallas import tpu as pltpu
from jax.experimental.pallas import tpu_sc as plsc

N = 10000
D = 128
E_EDGES = 320000
EPS = 1e-5
F32 = jnp.float32

NC = 2
NS = 16
NW = NC * NS
CHUNK = 128
CPW = 80
E_PAD = NW * CPW * CHUNK
ROWS_PER_SUB = 632
N_PAD = NS * ROWS_PER_SUB

_DEF = lax.Precision.DEFAULT


def _dot(a, b):
  return lax.dot(a, b, precision=_DEF, preferred_element_type=F32)


TBL_PER_SUB = N // NS


def _segsum_sc(ys, src_c, dst_c, zeros_pad, feat):
  n = len(ys)
  mesh = plsc.VectorSubcoreMesh(core_axis_name="c", subcore_axis_name="s")

  @functools.partial(
      pl.kernel,
      out_type=jax.ShapeDtypeStruct((n, NC, N_PAD, feat), F32),
      mesh=mesh,
      scratch_types=[
          pltpu.VMEM((CPW, CHUNK), jnp.int32),
          pltpu.VMEM((CPW, CHUNK), jnp.int32),
          pltpu.VMEM((2, CHUNK, feat), F32),
          pltpu.VMEM_SHARED((N_PAD, feat), F32),
          pltpu.VMEM_SHARED((N, feat), F32),
          pltpu.SemaphoreType.DMA,
          pltpu.SemaphoreType.DMA,
      ],
      compiler_params=pltpu.CompilerParams(use_tc_tiling_on_sc=False),
  )
  def seg_kernel(*refs):
    y_hbms = refs[:n]
    src_hbm, dst_hbm, zeros_hbm, out_hbm = refs[n:n + 4]
    src_v, dst_v, rows_v, acc_sh, tbl_sh, sem0, sem1 = refs[n + 4:]
    c = lax.axis_index("c")
    s = lax.axis_index("s")
    wid = c * NS + s
    pltpu.sync_copy(src_hbm.at[wid], src_v)
    pltpu.sync_copy(dst_hbm.at[wid], dst_v)
    t0 = s * TBL_PER_SUB
    r0 = s * ROWS_PER_SUB

    for k in range(n):
      pltpu.sync_copy(y_hbms[k].at[pl.ds(t0, TBL_PER_SUB)],
                      tbl_sh.at[pl.ds(t0, TBL_PER_SUB)])
      pltpu.sync_copy(zeros_hbm.at[pl.ds(r0, ROWS_PER_SUB)],
                      acc_sh.at[pl.ds(r0, ROWS_PER_SUB)])
      plsc.subcore_barrier()

      pltpu.async_copy(tbl_sh.at[src_v.at[0]], rows_v.at[0], sem0)

      def body(jj, carry):
        j0 = 2 * jj
        pltpu.async_copy(tbl_sh.at[src_v.at[j0 + 1]], rows_v.at[1], sem1)
        pltpu.make_async_copy(tbl_sh.at[src_v.at[j0]], rows_v.at[0],
                              sem0).wait()
        pltpu.sync_copy(rows_v.at[0], acc_sh.at[dst_v.at[j0]], add=True)

        @pl.when(j0 + 2 < CPW)
        def _():
          pltpu.async_copy(tbl_sh.at[src_v.at[j0 + 2]], rows_v.at[0], sem0)

        pltpu.make_async_copy(tbl_sh.at[src_v.at[j0 + 1]], rows_v.at[1],
                              sem1).wait()
        pltpu.sync_copy(rows_v.at[1], acc_sh.at[dst_v.at[j0 + 1]], add=True)
        return carry

      lax.fori_loop(0, CPW // 2, body, 0)
      plsc.subcore_barrier()
      pltpu.sync_copy(acc_sh.at[pl.ds(r0, ROWS_PER_SUB)],
                      out_hbm.at[k, c, pl.ds(r0, ROWS_PER_SUB)])

  return seg_kernel(*ys, src_c, dst_c, zeros_pad)


def _gcn_layer1(a0, a1, b0, b1, c0, c1, h, wrel_a, wrel_b, wrel_c,
                brel, wroot, g, be):
  def body(a0_ref, a1_ref, b0_ref, b1_ref, c0_ref, c1_ref, h_ref,
           wa_ref, wb_ref, wc_ref, brel_ref, wr_ref, g_ref, be_ref, o_ref):
    p = (_dot(a0_ref[...] + a1_ref[...], wa_ref[...])
         + _dot(b0_ref[...] + b1_ref[...], wb_ref[...])
         + _dot(c0_ref[...] + c1_ref[...], wc_ref[...])
         + brel_ref[...] + _dot(h_ref[...], wr_ref[...]))
    a = jnp.maximum(p, 0.0)
    m = jnp.mean(a, axis=0, keepdims=True)
    v = jnp.mean(a * a, axis=0, keepdims=True) - m * m
    o_ref[...] = (a - m) * (g_ref[...] * lax.rsqrt(v + EPS)) + be_ref[...]

  return pl.pallas_call(
      body,
      out_shape=jax.ShapeDtypeStruct((N, 32), F32),
  )(a0, a1, b0, b1, c0, c1, h, wrel_a, wrel_b, wrel_c, brel, wroot, g, be)


def _gcn_layer2_stats(a0, a1, h, wrel, brel, wroot, g, be, w1, b1):
  def body(a0_ref, a1_ref, h_ref, wl_ref, brel_ref, wr_ref, g_ref, be_ref,
           w1_ref, b1_ref, h2_ref, st_ref):
    agg = a0_ref[...] + a1_ref[...]
    p = _dot(agg, wl_ref[...]) + brel_ref[...] + _dot(h_ref[...], wr_ref[...])
    a = jnp.maximum(p, 0.0)
    m = jnp.mean(a, axis=0, keepdims=True)
    v = jnp.mean(a * a, axis=0, keepdims=True) - m * m
    h2 = (a - m) * (g_ref[...] * lax.rsqrt(v + EPS)) + be_ref[...]
    h2_ref[...] = h2
    blocks = []
    for k in range(8):
      sl = slice(256 * k, 256 * (k + 1))
      u = _dot(h2, w1_ref[:, sl]) + b1_ref[:, sl]
      blocks.append(jnp.concatenate(
          [jnp.sum(u, axis=0, keepdims=True),
           jnp.sum(u * u, axis=0, keepdims=True)], axis=0))
    st_ref[...] = jnp.concatenate(blocks, axis=1)

  return pl.pallas_call(
      body,
      out_shape=[
          jax.ShapeDtypeStruct((N, 32), F32),
          jax.ShapeDtypeStruct((2, 2048), F32),
      ],
  )(a0, a1, h, wrel, brel, wroot, g, be, w1, b1)


_R1 = 200
_T1 = N // _R1


def _recompute_bn_relu_matmul_stats(h2, w1, b1, st1, gn1, bn1, w2, b2):
  def body(h2_ref, w1_ref, b1_ref, st_ref, g_ref, be_ref, w2_ref, b2_ref,
           o_ref, so_ref, acc):
    i = pl.program_id(0)
    u1 = _dot(h2_ref[...], w1_ref[...]) + b1_ref[...]
    m = st_ref[0:1, :] * (1.0 / N)
    var = st_ref[1:2, :] * (1.0 / N) - m * m
    t = jnp.maximum((u1 - m) * (g_ref[...] * lax.rsqrt(var + EPS))
                    + be_ref[...], 0.0)
    o = _dot(t, w2_ref[...]) + b2_ref[...]
    o_ref[...] = o

    @pl.when(i == 0)
    def _():
      acc[...] = jnp.zeros_like(acc)

    acc[0:1, :] += jnp.sum(o, axis=0, keepdims=True)
    acc[1:2, :] += jnp.sum(o * o, axis=0, keepdims=True)
    so_ref[...] = acc[...]

  return pl.pallas_call(
      body,
      grid=(_T1,),
      in_specs=[
          pl.BlockSpec((_R1, 32), lambda i: (i, 0)),
          pl.BlockSpec((32, 2048), lambda i: (0, 0)),
          pl.BlockSpec((1, 2048), lambda i: (0, 0)),
          pl.BlockSpec((2, 2048), lambda i: (0, 0)),
          pl.BlockSpec((1, 2048), lambda i: (0, 0)),
          pl.BlockSpec((1, 2048), lambda i: (0, 0)),
          pl.BlockSpec((2048, 1024), lambda i: (0, 0)),
          pl.BlockSpec((1, 1024), lambda i: (0, 0)),
      ],
      out_specs=[
          pl.BlockSpec((_R1, 1024), lambda i: (i, 0)),
          pl.BlockSpec((2, 1024), lambda i: (0, 0)),
      ],
      out_shape=[
          jax.ShapeDtypeStruct((N, 1024), F32),
          jax.ShapeDtypeStruct((2, 1024), F32),
      ],
      scratch_shapes=[pltpu.VMEM((2, 1024), F32)],
  )(h2, w1, b1, st1, gn1, bn1, w2, b2)


def _stage3_tail(u2, st2, gn2, bn2, w3, b3, gn3, bn3, w4, b4):
  def body(u2_ref, st2_ref, g2_ref, be2_ref, w3_ref, b3_ref, g3_ref, be3_ref,
           w4_ref, b4_ref, o_ref, u3_scr, acc):
    p = pl.program_id(0)
    i = pl.program_id(1)

    @pl.when(jnp.logical_and(p == 0, i == 0))
    def _():
      acc[...] = jnp.zeros_like(acc)

    @pl.when(p == 0)
    def _():
      m = st2_ref[0:1, :] * (1.0 / N)
      var = st2_ref[1:2, :] * (1.0 / N) - m * m
      t = jnp.maximum((u2_ref[...] - m) * (g2_ref[...] * lax.rsqrt(var + EPS))
                      + be2_ref[...], 0.0)
      u3 = _dot(t, w3_ref[...]) + b3_ref[...]
      u3_scr[pl.ds(i * _R1, _R1), :] = u3
      acc[0:1, :] += jnp.sum(u3, axis=0, keepdims=True)
      acc[1:2, :] += jnp.sum(u3 * u3, axis=0, keepdims=True)

    @pl.when(p == 1)
    def _():
      m3 = acc[0:1, :] * (1.0 / N)
      v3 = acc[1:2, :] * (1.0 / N) - m3 * m3
      t3 = jnp.maximum(
          (u3_scr[pl.ds(i * _R1, _R1), :] - m3)
          * (g3_ref[...] * lax.rsqrt(v3 + EPS)) + be3_ref[...], 0.0)
      o_ref[...] = _dot(t3, w4_ref[...]) + b4_ref[...]

  return pl.pallas_call(
      body,
      grid=(2, _T1),
      in_specs=[
          pl.BlockSpec((_R1, 1024), lambda p, i: (jnp.where(p == 0, i, 0), 0)),
          pl.BlockSpec((2, 1024), lambda p, i: (0, 0)),
          pl.BlockSpec((1, 1024), lambda p, i: (0, 0)),
          pl.BlockSpec((1, 1024), lambda p, i: (0, 0)),
          pl.BlockSpec((1024, 32), lambda p, i: (0, 0)),
          pl.BlockSpec((1, 32), lambda p, i: (0, 0)),
          pl.BlockSpec((1, 32), lambda p, i: (0, 0)),
          pl.BlockSpec((1, 32), lambda p, i: (0, 0)),
          pl.BlockSpec((32, 1), lambda p, i: (0, 0)),
          pl.BlockSpec((1, 1), lambda p, i: (0, 0)),
      ],
      out_specs=pl.BlockSpec((_R1, 1), lambda p, i: (i, 0)),
      out_shape=jax.ShapeDtypeStruct((N, 1), F32),
      scratch_shapes=[
          pltpu.VMEM((N, 32), F32),
          pltpu.VMEM((2, 32), F32),
      ],
  )(u2, st2, gn2, bn2, w3, b3, gn3, bn3, w4, b4)


def kernel(x, edge_index, Wrel1, brel1, Wroot1, Wrel2, brel2, Wroot2,
           g1, be1, g2, be2, W1, b1, gn1, bn1, W2, b2, gn2, bn2,
           W3, b3, gn3, bn3, W4, b4):
  src = edge_index[0]
  dst = edge_index[1]
  npad = E_PAD - E_EDGES
  src_c = jnp.concatenate(
      [src, jnp.zeros((npad,), jnp.int32)]).reshape(NW, CPW, CHUNK)
  dst_c = jnp.concatenate(
      [dst, jnp.full((npad,), N, jnp.int32)]).reshape(NW, CPW, CHUNK)
  zeros48 = jnp.zeros((N_PAD, 48), F32)
  zeros32 = jnp.zeros((N_PAD, 32), F32)

  row = lambda t: t.reshape(1, -1)

  seg1ab = _segsum_sc([x[:, :48], x[:, 48:96]], src_c, dst_c, zeros48, 48)
  seg1c = _segsum_sc([x[:, 96:]], src_c, dst_c, zeros32, 32)
  h1 = _gcn_layer1(seg1ab[0, 0, :N], seg1ab[0, 1, :N],
                   seg1ab[1, 0, :N], seg1ab[1, 1, :N],
                   seg1c[0, 0, :N], seg1c[0, 1, :N],
                   x, Wrel1[:48], Wrel1[48:96], Wrel1[96:], row(brel1),
                   Wroot1, row(g1), row(be1))

  seg2 = _segsum_sc([h1], src_c, dst_c, zeros32, 32)
  h2, st1 = _gcn_layer2_stats(seg2[0, 0, :N], seg2[0, 1, :N], h1, Wrel2,
                              row(brel2), Wroot2, row(g2), row(be2),
                              W1, row(b1))

  u2, st2 = _recompute_bn_relu_matmul_stats(h2, W1, row(b1), st1,
                                            row(gn1), row(bn1), W2, row(b2))
  return _stage3_tail(u2, st2, row(gn2), row(bn2), W3, row(b3),
                      row(gn3), row(bn3), W4, row(b4))
